# speedup vs baseline: 1.6495x; 1.6495x over previous
import jax
import jax.numpy as jnp
from jax.experimental import pallas as pl


def _tc_body(seq_ref, idx_ref, tab_ref, out_ref):
    bb, _, l = seq_ref.shape
    v = tab_ref.shape[0]
    tab = tab_ref[...]
    idx2d = idx_ref[...]
    iota_v = jax.lax.broadcasted_iota(jnp.int32, (v, l), 0)
    for b in range(bb):
        onehot = (iota_v == idx2d[b][None, :]).astype(jnp.bfloat16)
        emb_t = jax.lax.dot_general(
            tab, onehot, (((0,), (0,)), ((), ())),
            preferred_element_type=jnp.float32)
        out_ref[b] = jnp.concatenate([seq_ref[b], emb_t], axis=1)


def kernel(sequence, time_index_sequence, variable_index_sequence, embed_table):
    del time_index_sequence
    b_total, e, l = sequence.shape
    v = embed_table.shape[0]
    bb = 8
    idx = variable_index_sequence.reshape(b_total, l).astype(jnp.int32)
    tab16 = embed_table.astype(jnp.bfloat16)
    return pl.pallas_call(
        _tc_body,
        grid=(b_total // bb,),
        in_specs=[
            pl.BlockSpec((bb, e, l), lambda i: (i, 0, 0)),
            pl.BlockSpec((bb, l), lambda i: (i, 0)),
            pl.BlockSpec((v, e), lambda i: (0, 0)),
        ],
        out_specs=pl.BlockSpec((bb, e, 2 * l), lambda i: (i, 0, 0)),
        out_shape=jax.ShapeDtypeStruct((b_total, e, 2 * l), jnp.float32),
    )(sequence, idx, tab16)
